# baseline (speedup 1.0000x reference)
import functools

import jax
import jax.numpy as jnp
from jax import lax
from jax.experimental import pallas as pl
from jax.experimental.pallas import tpu as pltpu
from jax.experimental.pallas import tpu_sc as plsc

_info = plsc.get_sparse_core_info()
_NC, _NS, _L = _info.num_cores, _info.num_subcores, _info.num_lanes
_NW = _NC * _NS

_CHUNK = 128


def _sc_gather(idx_t, table):
    hist, batch = idx_t.shape
    cpw = batch // (_NW * _CHUNK)
    bpw = cpw * _CHUNK
    nbc = batch // _CHUNK
    mesh = plsc.VectorSubcoreMesh(core_axis_name="c", subcore_axis_name="s")

    @functools.partial(
        pl.kernel,
        mesh=mesh,
        compiler_params=pltpu.CompilerParams(
            use_tc_tiling_on_sc=False, needs_layout_passes=False
        ),
        out_type=jax.ShapeDtypeStruct((hist, 4, nbc, 8, _CHUNK), jnp.float32),
        scratch_types=[
            pltpu.VMEM((hist, bpw), jnp.int32),
            pltpu.VMEM((4, _CHUNK, 32), jnp.float32),
            pltpu.VMEM((_CHUNK, 33), jnp.float32),
            pltpu.VMEM((2, 4, 8, _CHUNK), jnp.float32),
            pltpu.SemaphoreType.DMA,
            pltpu.SemaphoreType.DMA,
        ],
    )
    def k(idx_hbm, tab_hbm, out_hbm, idx_v, rows_v, skew_v, blk_v, gsem, osem):
        wid = lax.axis_index("s") * _NC + lax.axis_index("c")
        b_base = wid * bpw
        n_units = hist * cpw
        pltpu.sync_copy(idx_hbm.at[:, pl.ds(b_base, bpw)], idx_v)
        lanes = jax.lax.iota(jnp.int32, _L)
        rowv = [
            lanes + jnp.full((_L,), g * _L, jnp.int32)
            for g in range(_CHUNK // _L)
        ]
        colv = [jnp.full((_L,), c, jnp.int32) for c in range(32)]

        def fire_gather(u, buf):
            h = u // cpw
            j = u % cpw
            pltpu.async_copy(
                tab_hbm.at[idx_v.at[h, pl.ds(j * _CHUNK, _CHUNK)]],
                rows_v.at[buf],
                gsem,
            )

        for p in range(3):
            fire_gather(p, p)

        @pl.loop(0, n_units, step=4)
        def _pair(u0):
            for du in range(4):
                u = u0 + du
                buf = du
                obuf = du % 2
                pltpu.make_async_copy(
                    tab_hbm.at[pl.ds(0, _CHUNK)], rows_v.at[buf], gsem
                ).wait()

                @pl.when(u + 3 < n_units)
                def _():
                    fire_gather(u + 3, (buf + 3) % 4)

                @pl.when(u >= 2)
                def _():
                    pltpu.make_async_copy(
                        tab_hbm.at[pl.ds(0, 4 * 8 * _CHUNK // 32)],
                        blk_v.at[obuf],
                        osem,
                    ).wait()

                for b in range(_CHUNK):
                    for q in range(2):
                        skew_v[b, pl.ds(q * _L, _L)] = rows_v[
                            buf, b, pl.ds(q * _L, _L)
                        ]
                for c in range(32):
                    vals = [
                        plsc.load_gather(skew_v, [rowv[g], colv[c]])
                        for g in range(_CHUNK // _L)
                    ]
                    for g in range(_CHUNK // _L):
                        blk_v[obuf, c // 8, c % 8, pl.ds(g * _L, _L)] = vals[g]
                h = u // cpw
                j = u % cpw
                pltpu.async_copy(
                    blk_v.at[obuf], out_hbm.at[h, :, wid * cpw + j], osem
                )

        for buf in range(2):
            pltpu.make_async_copy(
                tab_hbm.at[pl.ds(0, 4 * 8 * _CHUNK // 32)], blk_v.at[buf], osem
            ).wait()

    return k(idx_t, table)


def kernel(indices, table):
    batch, hist = indices.shape
    vocab, dim = table.shape
    assert batch % (_NW * _CHUNK) == 0 and dim == 32
    out5 = _sc_gather(indices.astype(jnp.int32).T, table)
    return out5.transpose(2, 4, 0, 1, 3).reshape(batch, hist, dim)


# speedup vs baseline: 1.0924x; 1.0924x over previous
import functools

import jax
import jax.numpy as jnp
from jax import lax
from jax.experimental import pallas as pl
from jax.experimental.pallas import tpu as pltpu
from jax.experimental.pallas import tpu_sc as plsc

_info = plsc.get_sparse_core_info()
_NC, _NS, _L = _info.num_cores, _info.num_subcores, _info.num_lanes
_NW = _NC * _NS

_CHUNK = 128


def _sc_gather(idx_t, table):
    hist, batch = idx_t.shape
    cpw = batch // (_NW * _CHUNK)
    bpw = cpw * _CHUNK
    nbc = batch // _CHUNK
    mesh = plsc.VectorSubcoreMesh(core_axis_name="c", subcore_axis_name="s")

    @functools.partial(
        pl.kernel,
        mesh=mesh,
        compiler_params=pltpu.CompilerParams(
            use_tc_tiling_on_sc=False, needs_layout_passes=False
        ),
        out_type=jax.ShapeDtypeStruct((hist, 4, nbc, 8, _CHUNK), jnp.float32),
        scratch_types=[
            pltpu.VMEM((hist, bpw), jnp.int32),
            pltpu.VMEM((2, _CHUNK, 32), jnp.float32),
            pltpu.VMEM((_CHUNK, 33), jnp.float32),
            pltpu.VMEM((2, 4, 8, _CHUNK), jnp.float32),
            pltpu.SemaphoreType.DMA,
            pltpu.SemaphoreType.DMA,
        ],
    )
    def k(idx_hbm, tab_hbm, out_hbm, idx_v, rows_v, skew_v, blk_v, gsem, osem):
        wid = lax.axis_index("s") * _NC + lax.axis_index("c")
        b_base = wid * bpw
        n_units = hist * cpw
        pltpu.sync_copy(idx_hbm.at[:, pl.ds(b_base, bpw)], idx_v)
        lanes = jax.lax.iota(jnp.int32, _L)
        rowv = [
            lanes + jnp.full((_L,), g * _L, jnp.int32)
            for g in range(_CHUNK // _L)
        ]
        colv = [jnp.full((_L,), c, jnp.int32) for c in range(32)]

        def fire_gather(u, buf):
            h = u // cpw
            j = u % cpw
            pltpu.async_copy(
                tab_hbm.at[idx_v.at[h, pl.ds(j * _CHUNK, _CHUNK)]],
                rows_v.at[buf],
                gsem,
            )

        fire_gather(0, 0)

        @pl.loop(0, n_units, step=2)
        def _pair(u0):
            for du in range(2):
                u = u0 + du
                buf = du
                obuf = du
                pltpu.make_async_copy(
                    tab_hbm.at[pl.ds(0, _CHUNK)], rows_v.at[buf], gsem
                ).wait()

                @pl.when(u + 1 < n_units)
                def _():
                    fire_gather(u + 1, 1 - buf)

                @pl.when(u >= 2)
                def _():
                    pltpu.make_async_copy(
                        tab_hbm.at[pl.ds(0, 4 * 8 * _CHUNK // 32)],
                        blk_v.at[obuf],
                        osem,
                    ).wait()

                for b in range(_CHUNK):
                    for q in range(2):
                        skew_v[b, pl.ds(q * _L, _L)] = rows_v[
                            buf, b, pl.ds(q * _L, _L)
                        ]
                for c in range(32):
                    vals = [
                        plsc.load_gather(skew_v, [rowv[g], colv[c]])
                        for g in range(_CHUNK // _L)
                    ]
                    for g in range(_CHUNK // _L):
                        blk_v[obuf, c // 8, c % 8, pl.ds(g * _L, _L)] = vals[g]
                h = u // cpw
                j = u % cpw
                pltpu.async_copy(
                    blk_v.at[obuf], out_hbm.at[h, :, wid * cpw + j], osem
                )

        for buf in range(2):
            pltpu.make_async_copy(
                tab_hbm.at[pl.ds(0, 4 * 8 * _CHUNK // 32)], blk_v.at[buf], osem
            ).wait()

    return k(idx_t, table)


def kernel(indices, table):
    batch, hist = indices.shape
    vocab, dim = table.shape
    assert batch % (_NW * _CHUNK) == 0 and dim == 32
    out5 = _sc_gather(indices.astype(jnp.int32).T, table)
    return out5.transpose(2, 4, 0, 1, 3).reshape(batch, hist, dim)
